# Initial kernel scaffold; baseline (speedup 1.0000x reference)
import jax
import jax.numpy as jnp
from jax.experimental import pallas as pl


def kernel(X, edge_index, edge_type, l, w_bases, w_rel):
    raise NotImplementedError("write your pallas kernel here")


# speedup vs baseline: 9.6626x; 9.6626x over previous
import functools

import jax
import jax.numpy as jnp
from jax import lax
from jax.experimental import pallas as pl
from jax.experimental.pallas import tpu as pltpu
from jax.experimental.pallas import tpu_sc as plsc

NC = 2
NS = 16
L = 16
NW = NC * NS
CH = 128
EPS = 1e-5


def _cdiv(a, b):
    return -(-a // b)


def _pick_block(n, cap=1024):
    for bn in range(min(n, cap), 0, -1):
        if n % bn == 0 and (bn % 8 == 0 or bn == n):
            return bn
    return n


def _z_body(wrel_ref, x_ref, wb_ref, z_ref):
    r = pl.program_id(0)
    w = wrel_ref[r, 0] * wb_ref[0]
    for b in range(1, wb_ref.shape[0]):
        w += wrel_ref[r, b] * wb_ref[b]
    z_ref[0] = jnp.dot(x_ref[...], w, preferred_element_type=jnp.float32,
                       precision=lax.Precision.HIGHEST)


def _compute_z(X, w_bases, w_rel):
    n, d_in = X.shape
    r, b = w_rel.shape
    d_out = w_bases.shape[2]
    bn = _pick_block(n)
    return pl.pallas_call(
        _z_body,
        grid=(r, n // bn),
        in_specs=[
            pl.BlockSpec(memory_space=pltpu.SMEM),
            pl.BlockSpec((bn, d_in), lambda i, j: (j, 0)),
            pl.BlockSpec((b, d_in, d_out), lambda i, j: (0, 0, 0)),
        ],
        out_specs=pl.BlockSpec((1, bn, d_out), lambda i, j: (i, j, 0)),
        out_shape=jax.ShapeDtypeStruct((r, n, d_out), jnp.float32),
    )(w_rel, X, w_bases)


def _add_body(a_ref, b_ref, o_ref):
    o_ref[...] = a_ref[...] + b_ref[...]


def _combine(p0, p1):
    n, d = p0.shape
    bn = _pick_block(n)
    return pl.pallas_call(
        _add_body,
        grid=(n // bn,),
        in_specs=[pl.BlockSpec((bn, d), lambda i: (i, 0)),
                  pl.BlockSpec((bn, d), lambda i: (i, 0))],
        out_specs=pl.BlockSpec((bn, d), lambda i: (i, 0)),
        out_shape=jax.ShapeDtypeStruct((n, d), jnp.float32),
    )(p0, p1)


def _make_deg_kernel(nch1, deg_pad, r_rel):
    mesh = plsc.VectorSubcoreMesh(core_axis_name="c", subcore_axis_name="s")
    dsl = deg_pad // NS

    @functools.partial(
        pl.kernel, mesh=mesh,
        out_type=jax.ShapeDtypeStruct((deg_pad,), jnp.float32),
        scratch_types=[
            pltpu.VMEM((nch1, CH), jnp.int32),
            pltpu.VMEM((nch1, CH), jnp.int32),
            pltpu.VMEM((CH,), jnp.int32),
            pltpu.VMEM((CH,), jnp.float32),
            pltpu.VMEM((dsl,), jnp.float32),
            pltpu.VMEM_SHARED((deg_pad,), jnp.float32),
        ])
    def deg_kernel(row2_hbm, typ2_hbm, degr_hbm,
                   row_t, typ_t, didx_v, ones_v, wb_v, deg_sp):
        cid = lax.axis_index("c")
        sid = lax.axis_index("s")

        @pl.when(cid == 0)
        def _():
            zero = jnp.zeros((L,), jnp.float32)

            def zb(i, c):
                wb_v[pl.ds(i * L, L)] = zero
                return c
            lax.fori_loop(0, dsl // L, zb, None)
            d0 = sid * dsl
            pltpu.sync_copy(wb_v, deg_sp.at[pl.ds(d0, dsl)])
            one = jnp.ones((L,), jnp.float32)
            for j in range(CH // L):
                ones_v[pl.ds(j * L, L)] = one
            c0 = sid * nch1
            pltpu.sync_copy(row2_hbm.at[pl.ds(c0, nch1)], row_t)
            pltpu.sync_copy(typ2_hbm.at[pl.ds(c0, nch1)], typ_t)
            plsc.subcore_barrier()

            def body(k, c):
                for j in range(CH // L):
                    sl = pl.ds(j * L, L)
                    didx_v[sl] = row_t[k, sl] * r_rel + typ_t[k, sl]
                pltpu.sync_copy(ones_v, deg_sp.at[didx_v], add=True)
                return c
            lax.fori_loop(0, nch1, body, None)
            plsc.subcore_barrier()

            pltpu.sync_copy(deg_sp.at[pl.ds(d0, dsl)], wb_v)

            def rb(i, c):
                sl = pl.ds(i * L, L)
                wb_v[sl] = 1.0 / (wb_v[sl] + EPS)
                return c
            lax.fori_loop(0, dsl // L, rb, None)
            pltpu.sync_copy(wb_v, degr_hbm.at[pl.ds(d0, dsl)])

    return deg_kernel


def _make_main_kernel(nch, n_pad, r_rel, n_nodes, d):
    mesh = plsc.VectorSubcoreMesh(core_axis_name="c", subcore_axis_name="s")
    rpt = n_pad // NS

    @functools.partial(
        pl.kernel, mesh=mesh,
        out_type=(jax.ShapeDtypeStruct((n_pad, d), jnp.float32),
                  jax.ShapeDtypeStruct((n_pad, d), jnp.float32)),
        scratch_types=[
            pltpu.VMEM((nch, CH), jnp.int32),
            pltpu.VMEM((nch, CH), jnp.int32),
            pltpu.VMEM((nch, CH), jnp.int32),
            pltpu.VMEM((CH,), jnp.int32),
            pltpu.VMEM((CH,), jnp.int32),
            pltpu.VMEM((CH,), jnp.int32),
            pltpu.VMEM((CH + L,), jnp.float32),
            pltpu.VMEM((CH, d), jnp.float32),
            pltpu.VMEM_SHARED((n_pad, d), jnp.float32),
            pltpu.SemaphoreType.DMA,
        ])
    def main_kernel(row2_hbm, col2_hbm, typ2_hbm, z_hbm, degr_hbm,
                    p0_hbm, p1_hbm,
                    row_t, col_t, typ_t, row_v, didx_v, zidx_v, c_v, zbuf,
                    acc, sem):
        cid = lax.axis_index("c")
        sid = lax.axis_index("s")
        wid = cid * NS + sid

        zero = jnp.zeros((L,), jnp.float32)

        def zb(i, c):
            for j in range(d // L):
                zbuf[i, pl.ds(j * L, L)] = zero
            return c
        lax.fori_loop(0, CH, zb, None)
        base = sid * rpt
        off = 0
        while off < rpt:
            cnt = min(CH, rpt - off)
            pltpu.sync_copy(zbuf.at[pl.ds(0, cnt)],
                            acc.at[pl.ds(base + off, cnt)])
            off += cnt

        e0 = wid * nch
        pltpu.sync_copy(row2_hbm.at[pl.ds(e0, nch)], row_t)
        pltpu.sync_copy(col2_hbm.at[pl.ds(e0, nch)], col_t)
        pltpu.sync_copy(typ2_hbm.at[pl.ds(e0, nch)], typ_t)
        plsc.subcore_barrier()

        def body(k, carry):
            for j in range(CH // L):
                sl = pl.ds(j * L, L)
                r = row_t[k, sl]
                t = typ_t[k, sl]
                cc = col_t[k, sl]
                row_v[sl] = r
                didx_v[sl] = r * r_rel + t
                zidx_v[sl] = t * n_nodes + cc
            pltpu.sync_copy(degr_hbm.at[didx_v], c_v.at[pl.ds(0, CH)])
            pltpu.async_copy(z_hbm.at[zidx_v], zbuf, sem).wait()

            def srow(i, c2):
                cs = c_v[pl.ds(i, L)][0]
                for j in range(d // L):
                    sl2 = pl.ds(j * L, L)
                    zbuf[i, sl2] = zbuf[i, sl2] * cs
                return c2
            lax.fori_loop(0, CH, srow, None)
            pltpu.sync_copy(zbuf, acc.at[row_v], add=True)
            return carry
        lax.fori_loop(0, nch, body, None)
        plsc.subcore_barrier()

        @pl.when(cid == 0)
        def _():
            off = 0
            while off < rpt:
                cnt = min(CH, rpt - off)
                pltpu.sync_copy(acc.at[pl.ds(base + off, cnt)],
                                p0_hbm.at[pl.ds(base + off, cnt)])
                off += cnt

        @pl.when(cid == 1)
        def _():
            off = 0
            while off < rpt:
                cnt = min(CH, rpt - off)
                pltpu.sync_copy(acc.at[pl.ds(base + off, cnt)],
                                p1_hbm.at[pl.ds(base + off, cnt)])
                off += cnt

    return main_kernel


def kernel(X, edge_index, edge_type, l, w_bases, w_rel):
    del l
    n, _ = X.shape
    r_rel, _ = w_rel.shape
    d_out = w_bases.shape[2]
    e = edge_type.shape[0]

    e_pad = _cdiv(e, NW * CH * 8) * NW * CH * 8
    pad = e_pad - e
    row = jnp.concatenate([edge_index[0], jnp.full((pad,), n, jnp.int32)])
    col = jnp.concatenate([edge_index[1], jnp.zeros((pad,), jnp.int32)])
    typ = jnp.concatenate([edge_type, jnp.zeros((pad,), jnp.int32)])
    row2 = row.reshape(-1, CH)
    col2 = col.reshape(-1, CH)
    typ2 = typ.reshape(-1, CH)

    z = _compute_z(X, w_bases, w_rel)
    z2 = z.reshape(r_rel * n, d_out)

    deg_pad = _cdiv(n * r_rel + r_rel, NS * L) * NS * L
    n_pad = _cdiv(n + 1, NS * 8) * NS * 8
    nchunks = e_pad // CH
    degr = _make_deg_kernel(nchunks // NS, deg_pad, r_rel)(row2, typ2)
    p0, p1 = _make_main_kernel(nchunks // NW, n_pad, r_rel, n, d_out)(
        row2, col2, typ2, z2, degr)
    return _combine(p0[:n], p1[:n])
